# baseline (speedup 1.0000x reference)
import functools

import jax
import jax.numpy as jnp
from jax import lax
from jax.experimental import pallas as pl
from jax.experimental.pallas import tpu as pltpu
from jax.experimental.pallas import tpu_sc as plsc

_NBINS = 16
_NC, _NS, _L = 2, 16, 16
_NW = _NC * _NS
_CROWS = 16
_UNROLL = 8


def _pwla_call(nrows, ncols):
    rows_per_w = nrows // _NW
    n_chunks = rows_per_w // _CROWS
    ch = _CROWS * ncols

    mesh = plsc.VectorSubcoreMesh(
        core_axis_name="c", subcore_axis_name="s",
        num_cores=_NC, num_subcores=_NS)

    @functools.partial(
        pl.kernel,
        out_type=jax.ShapeDtypeStruct((nrows, ncols), jnp.float32),
        mesh=mesh,
        compiler_params=pltpu.CompilerParams(needs_layout_passes=False),
        scratch_types=[
            pltpu.VMEM((_CROWS, ncols), jnp.float32),
            pltpu.VMEM((_CROWS, ncols), jnp.float32),
            pltpu.VMEM((_CROWS, ncols), jnp.float32),
            pltpu.VMEM((32,), jnp.float32),
            pltpu.VMEM((32,), jnp.float32),
            pltpu.VMEM((_L,), jnp.float32),
            pltpu.VMEM((_L,), jnp.float32),
            pltpu.SemaphoreType.DMA,
            pltpu.SemaphoreType.DMA,
            pltpu.SemaphoreType.DMA,
            pltpu.SemaphoreType.DMA,
            pltpu.SemaphoreType.DMA,
            pltpu.SemaphoreType.DMA,
        ],
    )
    def k(x_hbm, a_hbm, k_hbm, sc_hbm, sh_hbm, out_hbm,
          buf0, buf1, buf2, a_v, k_v, sc_v, sh_v,
          isem0, isem1, isem2, osem0, osem1, osem2):
        wid = lax.axis_index("s") * _NC + lax.axis_index("c")
        base_row = wid * rows_per_w

        bufs = (buf0, buf1, buf2)
        isems = (isem0, isem1, isem2)
        osems = (osem0, osem1, osem2)

        def in_copy(c, b):
            return pltpu.make_async_copy(
                x_hbm.at[pl.ds(base_row + c * _CROWS, _CROWS), :],
                bufs[b], isems[b])

        def out_copy(c, b):
            return pltpu.make_async_copy(
                bufs[b],
                out_hbm.at[pl.ds(base_row + c * _CROWS, _CROWS), :],
                osems[b])

        col_shift = ncols.bit_length() - 1

        def compute(b, scale, shift):
            buf_v = bufs[b]

            @plsc.parallel_loop(0, ch, step=_L, unroll=_UNROLL)
            def vbody(off):
                r = lax.shift_right_logical(off, col_shift)
                col = lax.bitwise_and(off, ncols - 1)
                xv = buf_v[r, pl.ds(col, _L)]
                t = jnp.minimum(
                    jnp.maximum(xv * scale + shift, 0.0),
                    float(_NBINS + 1))
                i = t.astype(jnp.int32)
                av = plsc.load_gather(a_v, [i])
                kv = plsc.load_gather(k_v, [i])
                buf_v[r, pl.ds(col, _L)] = av + xv * kv

        in_copy(0, 0).start()
        in_copy(1, 1).start()
        in_copy(2, 2).start()
        pltpu.sync_copy(a_hbm, a_v)
        pltpu.sync_copy(k_hbm, k_v)
        pltpu.sync_copy(sc_hbm, sc_v)
        pltpu.sync_copy(sh_hbm, sh_v)
        scale = sc_v[...]
        shift = sh_v[...]
        compute = functools.partial(compute, scale=scale, shift=shift)

        in_copy(0, 0).wait()
        compute(0)
        out_copy(0, 0).start()

        def trip(p, carry):
            for q in range(3):
                c = 3 * p + 1 + q
                b = (1 + q) % 3
                in_copy(c, b).wait()
                compute(b)
                out_copy(c, b).start()
                out_copy(c - 1, (b + 2) % 3).wait()

                @pl.when(c + 2 < n_chunks)
                def _():
                    in_copy(c + 2, (b + 2) % 3).start()

            return carry

        lax.fori_loop(0, (n_chunks - 1) // 3, trip, 0)
        out_copy(n_chunks - 1, (n_chunks - 1) % 3).wait()

    return k


def kernel(x, mode, Br, Bl, Kl, Kr, Yidx):
    del mode
    orig_shape = x.shape
    ncols = x.shape[-1]
    x2 = x.reshape(-1, ncols)
    nrows = x2.shape[0]

    f32 = jnp.float32
    Br = Br.astype(f32)
    Bl = Bl.astype(f32)
    inv_d = _NBINS / (Br - Bl)
    d = (Br - Bl) / _NBINS

    j = jnp.arange(_NBINS, dtype=f32)
    k_in = (Yidx[1:] - Yidx[:-1]) * inv_d
    b_j = Bl + j * d
    a_in = Yidx[:-1] - b_j * k_in
    k_full = jnp.concatenate(
        [Kl[None].astype(f32), k_in, Kr[None].astype(f32)])
    a_full = jnp.concatenate(
        [(Yidx[0] - Bl * Kl)[None], a_in, (Yidx[-1] - Br * Kr)[None]])
    a_tab = jnp.zeros((32,), f32).at[: _NBINS + 2].set(a_full)
    k_tab = jnp.zeros((32,), f32).at[: _NBINS + 2].set(k_full)

    scale_vec = jnp.full((_L,), inv_d, f32)
    shift_vec = jnp.full((_L,), 1.0 - Bl * inv_d, f32)

    out = _pwla_call(nrows, ncols)(x2, a_tab, k_tab, scale_vec, shift_vec)
    return out.reshape(orig_shape)


# speedup vs baseline: 1.0425x; 1.0425x over previous
import functools

import jax
import jax.numpy as jnp
from jax import lax
from jax.experimental import pallas as pl
from jax.experimental.pallas import tpu as pltpu
from jax.experimental.pallas import tpu_sc as plsc

_NBINS = 16
_NC, _NS, _L = 2, 16, 16
_NW = _NC * _NS
_CROWS = 8
_UNROLL = 8


def _pwla_call(nrows, ncols):
    rows_per_w = nrows // _NW
    n_chunks = rows_per_w // _CROWS
    ch = _CROWS * ncols

    mesh = plsc.VectorSubcoreMesh(
        core_axis_name="c", subcore_axis_name="s",
        num_cores=_NC, num_subcores=_NS)

    @functools.partial(
        pl.kernel,
        out_type=jax.ShapeDtypeStruct((nrows, ncols), jnp.float32),
        mesh=mesh,
        compiler_params=pltpu.CompilerParams(needs_layout_passes=False),
        scratch_types=[
            pltpu.VMEM((_CROWS, ncols), jnp.float32),
            pltpu.VMEM((_CROWS, ncols), jnp.float32),
            pltpu.VMEM((_CROWS, ncols), jnp.float32),
            pltpu.VMEM((_CROWS, ncols), jnp.float32),
            pltpu.VMEM((96,), jnp.float32),
            pltpu.SemaphoreType.DMA,
            pltpu.SemaphoreType.DMA,
            pltpu.SemaphoreType.DMA,
            pltpu.SemaphoreType.DMA,
        ],
    )
    def k(x_hbm, tab_hbm, out_hbm,
          xin0, xin1, yout0, yout1, tab_v,
          isem0, isem1, osem0, osem1):
        wid = lax.axis_index("s") * _NC + lax.axis_index("c")
        base_row = wid * rows_per_w

        xbufs = (xin0, xin1)
        ybufs = (yout0, yout1)
        isems = (isem0, isem1)
        osems = (osem0, osem1)

        def in_copy(c, b):
            return pltpu.make_async_copy(
                x_hbm.at[pl.ds(base_row + c * _CROWS, _CROWS), :],
                xbufs[b], isems[b])

        def out_copy(c, b):
            return pltpu.make_async_copy(
                ybufs[b],
                out_hbm.at[pl.ds(base_row + c * _CROWS, _CROWS), :],
                osems[b])

        col_shift = ncols.bit_length() - 1

        def compute(b, scale, shift):
            xin_v = xbufs[b]
            yout_v = ybufs[b]

            @plsc.parallel_loop(0, ch, step=_L, unroll=_UNROLL)
            def vbody(off):
                r = lax.shift_right_logical(off, col_shift)
                col = lax.bitwise_and(off, ncols - 1)
                xv = xin_v[r, pl.ds(col, _L)]
                t = jnp.minimum(
                    jnp.maximum(xv * scale + shift, 0.0),
                    float(_NBINS + 1))
                i = t.astype(jnp.int32)
                av = plsc.load_gather(tab_v.at[pl.ds(0, 32)], [i])
                kv = plsc.load_gather(tab_v.at[pl.ds(32, 32)], [i])
                yout_v[r, pl.ds(col, _L)] = av + xv * kv

        in_copy(0, 0).start()
        in_copy(1, 1).start()
        pltpu.sync_copy(tab_hbm, tab_v)
        scale = tab_v[pl.ds(64, _L)]
        shift = tab_v[pl.ds(80, _L)]
        compute = functools.partial(compute, scale=scale, shift=shift)
        for b in range(2):
            in_copy(b, b).wait()
            compute(b)
            out_copy(b, b).start()
            in_copy(b + 2, b).start()

        def pair(p, carry):
            for b in range(2):
                c = 2 * p + b
                in_copy(c, b).wait()
                out_copy(c - 2, b).wait()
                compute(b)
                out_copy(c, b).start()
                in_copy(c + 2, b).start()
            return carry

        lax.fori_loop(1, n_chunks // 2 - 1, pair, 0)

        for b in range(2):
            c = n_chunks - 2 + b
            in_copy(c, b).wait()
            out_copy(c - 2, b).wait()
            compute(b)
            out_copy(c, b).start()
        for b in range(2):
            out_copy(n_chunks - 2 + b, b).wait()

    return k


def kernel(x, mode, Br, Bl, Kl, Kr, Yidx):
    del mode
    orig_shape = x.shape
    ncols = x.shape[-1]
    x2 = x.reshape(-1, ncols)
    nrows = x2.shape[0]

    f32 = jnp.float32
    Br = Br.astype(f32)
    Bl = Bl.astype(f32)
    inv_d = _NBINS / (Br - Bl)
    d = (Br - Bl) / _NBINS

    j = jnp.arange(_NBINS, dtype=f32)
    k_in = (Yidx[1:] - Yidx[:-1]) * inv_d
    b_j = Bl + j * d
    a_in = Yidx[:-1] - b_j * k_in
    k_full = jnp.concatenate(
        [Kl[None].astype(f32), k_in, Kr[None].astype(f32)])
    a_full = jnp.concatenate(
        [(Yidx[0] - Bl * Kl)[None], a_in, (Yidx[-1] - Br * Kr)[None]])
    tab = jnp.zeros((96,), f32)
    tab = tab.at[: _NBINS + 2].set(a_full)
    tab = tab.at[32 : 32 + _NBINS + 2].set(k_full)
    tab = tab.at[64:80].set(jnp.full((_L,), inv_d, f32))
    tab = tab.at[80:96].set(jnp.full((_L,), 1.0 - Bl * inv_d, f32))

    out = _pwla_call(nrows, ncols)(x2, tab)
    return out.reshape(orig_shape)
